# baseline (speedup 1.0000x reference)
import functools

import jax
import jax.numpy as jnp
from jax import lax
from jax.experimental import pallas as pl
from jax.experimental.pallas import tpu as pltpu
from jax.experimental.pallas import tpu_sc as plsc

BS, W, D, NSAMP = 16384, 10, 64, 2
N = BS * W * NSAMP
NWORK = 32
PER = N // NWORK
BPT = BS // NWORK
CH = 1024
NCH = PER // CH
SUB = 128
NSUB = CH // SUB
SPW = W * NSAMP

_mesh = plsc.VectorSubcoreMesh(core_axis_name="c", subcore_axis_name="s")


@functools.partial(
    pl.kernel,
    mesh=_mesh,
    out_type=jax.ShapeDtypeStruct((N,), jnp.float32),
    compiler_params=pltpu.CompilerParams(
        needs_layout_passes=False, use_tc_tiling_on_sc=False
    ),
    scratch_types=[
        pltpu.VMEM((NSUB, SUB), jnp.int32),
        pltpu.VMEM((CH, D), jnp.float32),
        pltpu.VMEM((BPT, D), jnp.float32),
        pltpu.VMEM((CH,), jnp.float32),
        pltpu.SemaphoreType.DMA,
    ],
)
def _sc_dots(emb_hbm, cent_hbm, sidx_hbm, out_hbm, idx_v, rows_v, cent_v,
             dots_v, sem):
    cid = lax.axis_index("c")
    sid = lax.axis_index("s")
    wid = sid * 2 + cid
    base = wid * PER
    lane = lax.broadcasted_iota(jnp.int32, (16,), 0)

    pltpu.sync_copy(cent_hbm.at[pl.ds(wid * BPT, BPT)], cent_v)

    def chunk_body(ci, carry):
        s0 = base + ci * CH
        row0 = pl.multiple_of(s0 // SUB, NSUB)
        pltpu.sync_copy(sidx_hbm.at[pl.ds(row0, NSUB)], idx_v)
        cps = [
            pltpu.async_copy(emb_hbm.at[idx_v.at[j]],
                             rows_v.at[pl.ds(j * SUB, SUB)], sem)
            for j in range(NSUB)
        ]
        for cp in cps:
            cp.wait()

        def grp(g, c2):
            lid = g * 16 + lane
            l_all = ci * CH + lid
            b_ids = lax.div(l_all, jnp.full((16,), SPW, jnp.int32))
            acc = jnp.zeros((16,), jnp.float32)
            for k in range(D):
                kk = jnp.full((16,), k, jnp.int32)
                r = plsc.load_gather(rows_v, [lid, kk])
                c = plsc.load_gather(cent_v, [b_ids, kk])
                acc = acc + r * c
            plsc.store_scatter(dots_v, [lid], acc)
            return c2

        lax.fori_loop(0, CH // 16, grp, 0)
        pltpu.sync_copy(dots_v, out_hbm.at[pl.ds(s0, CH)])
        return carry

    lax.fori_loop(0, NCH, chunk_body, 0)


def _softplus(x):
    return jnp.maximum(x, 0.0) + jnp.log1p(jnp.exp(-jnp.abs(x)))


BLK = 1024
GRID = BS // BLK


def _tc_loss_body(ctx_ref, cent_ref, dots_ref, out_ref):
    i = pl.program_id(0)
    c = cent_ref[...]
    ce = jnp.broadcast_to(c[:, None, :], (BLK, W, D)).reshape(BLK * W, D)
    ts = jnp.sum(ctx_ref[...] * ce, axis=1, keepdims=True)
    part = jnp.sum(_softplus(-ts)) / (BS * W)
    part = part + jnp.sum(_softplus(dots_ref[...])) / N

    @pl.when(i == 0)
    def _():
        out_ref[0, 0] = 0.0

    out_ref[0, 0] += part


def _tc_loss(ctx_flat, center, dots2d):
    return pl.pallas_call(
        _tc_loss_body,
        grid=(GRID,),
        in_specs=[
            pl.BlockSpec((BLK * W, D), lambda i: (i, 0)),
            pl.BlockSpec((BLK, D), lambda i: (i, 0)),
            pl.BlockSpec((BLK, SPW), lambda i: (i, 0)),
        ],
        out_specs=pl.BlockSpec((1, 1), lambda i: (0, 0),
                               memory_space=pltpu.SMEM),
        out_shape=jax.ShapeDtypeStruct((1, 1), jnp.float32),
    )(ctx_flat, center, dots2d)


def kernel(center, context, emb_table, sample_idx):
    sidx = sample_idx.reshape(N // SUB, SUB).astype(jnp.int32)
    dots = _sc_dots(emb_table, center, sidx)
    ctx_flat = context.reshape(BS * W, D)
    dots2d = dots.reshape(BS, SPW)
    out = _tc_loss(ctx_flat, center, dots2d)
    return out[0, 0]


# speedup vs baseline: 1.3876x; 1.3876x over previous
import functools

import jax
import jax.numpy as jnp
from jax import lax
from jax.experimental import pallas as pl
from jax.experimental.pallas import tpu as pltpu
from jax.experimental.pallas import tpu_sc as plsc

BS, W, D, NSAMP = 16384, 10, 64, 2
N = BS * W * NSAMP
NWORK = 32
PER = N // NWORK
BPT = BS // NWORK
SPW = W * NSAMP
CH = 640
NCH = PER // CH
SUB = 128
NSUB = CH // SUB
IDXROWS = PER // SUB
BPC = CH // SPW
PAD = 17

_mesh = plsc.VectorSubcoreMesh(core_axis_name="c", subcore_axis_name="s")


@functools.partial(
    pl.kernel,
    mesh=_mesh,
    out_type=jax.ShapeDtypeStruct((N,), jnp.float32),
    compiler_params=pltpu.CompilerParams(
        needs_layout_passes=False, use_tc_tiling_on_sc=False
    ),
    scratch_types=[
        pltpu.VMEM((IDXROWS, SUB), jnp.int32),
        pltpu.VMEM((CH, D), jnp.float32),
        pltpu.VMEM((BPT, D), jnp.float32),
        pltpu.VMEM((CH, PAD), jnp.float32),
        pltpu.VMEM((CH,), jnp.float32),
        pltpu.SemaphoreType.DMA,
    ],
)
def _sc_dots(emb_hbm, cent_hbm, sidx_hbm, out_hbm, idx_v, rows_v, cent_v,
             part_v, dots_v, sem):
    cid = lax.axis_index("c")
    sid = lax.axis_index("s")
    wid = sid * 2 + cid
    base = wid * PER
    lane = lax.broadcasted_iota(jnp.int32, (16,), 0)

    pltpu.sync_copy(cent_hbm.at[pl.ds(wid * BPT, BPT)], cent_v)
    pltpu.sync_copy(sidx_hbm.at[pl.ds(wid * IDXROWS, IDXROWS)], idx_v)

    def chunk_body(ci, carry):
        s0 = base + ci * CH
        cps = [
            pltpu.async_copy(emb_hbm.at[idx_v.at[ci * NSUB + j]],
                             rows_v.at[pl.ds(j * SUB, SUB)], sem)
            for j in range(NSUB)
        ]
        for cp in cps:
            cp.wait()

        def phase_a(bb, c2):
            b = ci * BPC + bb
            cvs = [cent_v[b, pl.ds(kk * 16, 16)] for kk in range(D // 16)]
            for j in range(SPW):
                s = bb * SPW + j
                acc = rows_v[s, pl.ds(0, 16)] * cvs[0]
                for kk in range(1, D // 16):
                    acc = acc + rows_v[s, pl.ds(kk * 16, 16)] * cvs[kk]
                part_v[s, pl.ds(0, 16)] = acc
            return c2

        lax.fori_loop(0, BPC, phase_a, 0)

        def phase_b(g, c2):
            rows16 = g * 16 + lane
            acc = plsc.load_gather(part_v, [rows16, jnp.zeros((16,), jnp.int32)])
            for k in range(1, 16):
                acc = acc + plsc.load_gather(
                    part_v, [rows16, jnp.full((16,), k, jnp.int32)])
            plsc.store_scatter(dots_v, [rows16], acc)
            return c2

        lax.fori_loop(0, CH // 16, phase_b, 0)
        pltpu.sync_copy(dots_v, out_hbm.at[pl.ds(s0, CH)])
        return carry

    lax.fori_loop(0, NCH, chunk_body, 0)


def _softplus(x):
    return jnp.maximum(x, 0.0) + jnp.log1p(jnp.exp(-jnp.abs(x)))


BLK = 1024
GRID = BS // BLK


def _tc_loss_body(ctx_ref, cent_ref, dots_ref, out_ref):
    i = pl.program_id(0)
    c = cent_ref[...]
    part = jnp.sum(_softplus(dots_ref[...])) / N
    for w in range(W):
        ts = jnp.sum(ctx_ref[:, w, :] * c, axis=1, keepdims=True)
        part = part + jnp.sum(_softplus(-ts)) / (BS * W)

    @pl.when(i == 0)
    def _():
        out_ref[0, 0] = 0.0

    out_ref[0, 0] += part


def _tc_loss(context, center, dots2d):
    return pl.pallas_call(
        _tc_loss_body,
        grid=(GRID,),
        in_specs=[
            pl.BlockSpec((BLK, W, D), lambda i: (i, 0, 0)),
            pl.BlockSpec((BLK, D), lambda i: (i, 0)),
            pl.BlockSpec((BLK, SPW), lambda i: (i, 0)),
        ],
        out_specs=pl.BlockSpec((1, 1), lambda i: (0, 0),
                               memory_space=pltpu.SMEM),
        out_shape=jax.ShapeDtypeStruct((1, 1), jnp.float32),
    )(context, center, dots2d)


def kernel(center, context, emb_table, sample_idx):
    sidx = sample_idx.reshape(N // SUB, SUB).astype(jnp.int32)
    dots = _sc_dots(emb_table, center, sidx)
    dots2d = dots.reshape(BS, SPW)
    out = _tc_loss(context, center, dots2d)
    return out[0, 0]
